# baseline (speedup 1.0000x reference)
import functools

import jax
import jax.numpy as jnp
from jax import lax
from jax.experimental import pallas as pl
from jax.experimental.pallas import tpu as pltpu
from jax.experimental.pallas import tpu_sc as plsc

NC = 2
NS = 16
L = 16
NP = 3


def _sc_body(flat_hbm, bounds_hbm, starts_hbm, psum_hbm, cls_hbm,
             bounds_v, sidx_v, buf0, buf1, acc_v, cu_s,
             sem0, sem1, semc, *, T, D, S, R):
    cid = lax.axis_index("c")
    sid = lax.axis_index("s")
    wid = cid * NS + sid
    wch = T // (NC * NS)
    nchunk = wch // R
    fpp = D // NP
    npj = fpp // L
    base = wid * wch

    pltpu.sync_copy(bounds_hbm, bounds_v)
    bvals = bounds_v[...]
    lane = lax.broadcasted_iota(jnp.int32, (L,), 0)
    cu_s[0] = jnp.int32(0)
    for s in range(S):
        cu_s[s + 1] = jnp.sum(jnp.where(lane == s, bvals, 0))

    for r in range(S):
        def zb(j, carry):
            acc_v[r, pl.ds(j * L, L)] = jnp.zeros((L,), jnp.float32)
            return carry
        lax.fori_loop(0, D // L, zb, 0)

    def process(buf, clo):
        def seg_body(si, carry):
            glo = cu_s[si]
            ghi = cu_s[si + 1]
            lo = jnp.minimum(jnp.maximum(glo, clo), clo + R) - clo
            hi = jnp.minimum(jnp.maximum(ghi, clo), clo + R) - clo

            @pl.when(hi > lo)
            def _():
                for p in range(NP):
                    def rbody(r, carr):
                        return tuple(
                            carr[j] + buf[r, pl.ds(p * fpp + j * L, L)]
                            for j in range(npj))
                    zeros = tuple(
                        jnp.zeros((L,), jnp.float32) for _ in range(npj))
                    carr = lax.fori_loop(lo, hi, rbody, zeros)
                    for j in range(npj):
                        off = p * fpp + j * L
                        acc_v[si, pl.ds(off, L)] = (
                            acc_v[si, pl.ds(off, L)] + carr[j])
            return carry
        lax.fori_loop(0, S, seg_body, 0)

    pltpu.async_copy(flat_hbm.at[pl.ds(base, R)], buf0, sem0)
    pltpu.async_copy(flat_hbm.at[pl.ds(base + R, R)], buf1, sem1)

    def pair_body(k, carry):
        c0 = 2 * k
        clo0 = base + c0 * R
        pltpu.make_async_copy(flat_hbm.at[pl.ds(0, R)], buf0, sem0).wait()
        process(buf0, clo0)

        @pl.when(c0 + 2 < nchunk)
        def _():
            pltpu.async_copy(
                flat_hbm.at[pl.ds(clo0 + 2 * R, R)], buf0, sem0)

        clo1 = clo0 + R
        pltpu.make_async_copy(flat_hbm.at[pl.ds(0, R)], buf1, sem1).wait()
        process(buf1, clo1)

        @pl.when(c0 + 3 < nchunk)
        def _():
            pltpu.async_copy(
                flat_hbm.at[pl.ds(clo1 + 2 * R, R)], buf1, sem1)
        return carry

    lax.fori_loop(0, nchunk // 2, pair_body, 0)

    pltpu.sync_copy(acc_v, psum_hbm.at[wid])

    @pl.when((cid == 0) & (sid == 0))
    def _():
        pltpu.sync_copy(starts_hbm, sidx_v)
        pltpu.async_copy(flat_hbm.at[sidx_v], buf0.at[pl.ds(0, S)], semc).wait()
        pltpu.sync_copy(buf0.at[pl.ds(0, S)], cls_hbm)


def _head_body(psum_ref, cls_ref, invc_ref, W_ref, b_ref, out_ref):
    x = psum_ref[...]
    sums = jnp.sum(x, axis=0)
    mean = sums * invc_ref[...]
    pooled = jnp.concatenate([cls_ref[...], mean], axis=-1)
    out_ref[...] = (
        jnp.dot(pooled, W_ref[...], preferred_element_type=jnp.float32)
        + b_ref[...]
    )


def kernel(flat, cu_seqlens, W, b):
    T, D = flat.shape
    S = cu_seqlens.shape[0] - 1
    NL = W.shape[1]
    R = 64
    NW = NC * NS

    bounds = cu_seqlens[1:]
    starts = cu_seqlens[:-1]
    counts = (cu_seqlens[1:] - cu_seqlens[:-1]).astype(jnp.float32)
    invc = (1.0 / jnp.maximum(counts, 1.0)).reshape(S, 1)
    b2 = b.reshape(1, NL)

    mesh = plsc.VectorSubcoreMesh(
        core_axis_name="c", subcore_axis_name="s",
        num_cores=NC, num_subcores=NS)
    sc = functools.partial(
        pl.kernel,
        out_type=[
            jax.ShapeDtypeStruct((NW, S, D), jnp.float32),
            jax.ShapeDtypeStruct((S, D), jnp.float32),
        ],
        mesh=mesh,
        compiler_params=pltpu.CompilerParams(needs_layout_passes=False),
        scratch_types=[
            pltpu.VMEM((S,), jnp.int32),
            pltpu.VMEM((S,), jnp.int32),
            pltpu.VMEM((R, D), jnp.float32),
            pltpu.VMEM((R, D), jnp.float32),
            pltpu.VMEM((S, D), jnp.float32),
            pltpu.SMEM((S + 1,), jnp.int32),
            pltpu.SemaphoreType.DMA,
            pltpu.SemaphoreType.DMA,
            pltpu.SemaphoreType.DMA,
        ],
    )(functools.partial(_sc_body, T=T, D=D, S=S, R=R))
    psum, cls = sc(flat, bounds, starts)

    out = pl.pallas_call(
        _head_body,
        in_specs=[
            pl.BlockSpec((NW, S, D), lambda: (0, 0, 0)),
            pl.BlockSpec((S, D), lambda: (0, 0)),
            pl.BlockSpec((S, 1), lambda: (0, 0)),
            pl.BlockSpec((2 * D, NL), lambda: (0, 0)),
            pl.BlockSpec((1, NL), lambda: (0, 0)),
        ],
        out_specs=pl.BlockSpec((S, NL), lambda: (0, 0)),
        out_shape=jax.ShapeDtypeStruct((S, NL), jnp.float32),
    )(psum, cls, invc, W, b2)
    return out


# speedup vs baseline: 1.2032x; 1.2032x over previous
import functools

import jax
import jax.numpy as jnp
from jax import lax
from jax.experimental import pallas as pl
from jax.experimental.pallas import tpu as pltpu
from jax.experimental.pallas import tpu_sc as plsc

NC = 2
NS = 16
L = 16
NP = 3

TC_ROWS = 20480
TC_BLK = 2048


def _sc_body(flat_hbm, bounds_hbm, starts_hbm, psum_hbm, cls_hbm,
             bounds_v, sidx_v, buf0, buf1, acc_v, cu_s,
             sem0, sem1, semc, *, base0, T, D, S, R):
    cid = lax.axis_index("c")
    sid = lax.axis_index("s")
    wid = cid * NS + sid
    wch = (T - base0) // (NC * NS)
    nchunk = wch // R
    fpp = D // NP
    npj = fpp // L
    base = base0 + wid * wch

    pltpu.sync_copy(bounds_hbm, bounds_v)
    bvals = bounds_v[...]
    lane = lax.broadcasted_iota(jnp.int32, (L,), 0)
    cu_s[0] = jnp.int32(0)
    for s in range(S):
        cu_s[s + 1] = jnp.sum(jnp.where(lane == s, bvals, 0))

    for r in range(S):
        def zb(j, carry):
            acc_v[r, pl.ds(j * L, L)] = jnp.zeros((L,), jnp.float32)
            return carry
        lax.fori_loop(0, D // L, zb, 0)

    def process(buf, clo):
        def seg_body(si, carry):
            glo = cu_s[si]
            ghi = cu_s[si + 1]
            lo = jnp.minimum(jnp.maximum(glo, clo), clo + R) - clo
            hi = jnp.minimum(jnp.maximum(ghi, clo), clo + R) - clo

            @pl.when(hi > lo)
            def _():
                for p in range(NP):
                    def rbody(r, carr):
                        return tuple(
                            carr[j] + buf[r, pl.ds(p * fpp + j * L, L)]
                            for j in range(npj))
                    zeros = tuple(
                        jnp.zeros((L,), jnp.float32) for _ in range(npj))
                    carr = lax.fori_loop(lo, hi, rbody, zeros)
                    for j in range(npj):
                        off = p * fpp + j * L
                        acc_v[si, pl.ds(off, L)] = (
                            acc_v[si, pl.ds(off, L)] + carr[j])
            return carry
        lax.fori_loop(0, S, seg_body, 0)

    pltpu.async_copy(flat_hbm.at[pl.ds(base, R)], buf0, sem0)
    pltpu.async_copy(flat_hbm.at[pl.ds(base + R, R)], buf1, sem1)

    def pair_body(k, carry):
        c0 = 2 * k
        clo0 = base + c0 * R
        pltpu.make_async_copy(flat_hbm.at[pl.ds(0, R)], buf0, sem0).wait()
        process(buf0, clo0)

        @pl.when(c0 + 2 < nchunk)
        def _():
            pltpu.async_copy(
                flat_hbm.at[pl.ds(clo0 + 2 * R, R)], buf0, sem0)

        clo1 = clo0 + R
        pltpu.make_async_copy(flat_hbm.at[pl.ds(0, R)], buf1, sem1).wait()
        process(buf1, clo1)

        @pl.when(c0 + 3 < nchunk)
        def _():
            pltpu.async_copy(
                flat_hbm.at[pl.ds(clo1 + 2 * R, R)], buf1, sem1)
        return carry

    lax.fori_loop(0, nchunk // 2, pair_body, 0)

    pltpu.sync_copy(acc_v, psum_hbm.at[wid])

    @pl.when((cid == 0) & (sid == 0))
    def _():
        pltpu.sync_copy(starts_hbm, sidx_v)
        pltpu.async_copy(flat_hbm.at[sidx_v], buf0.at[pl.ds(0, S)], semc).wait()
        pltpu.sync_copy(buf0.at[pl.ds(0, S)], cls_hbm)


def _tc_body(starts_ref, ends_ref, x_ref, out_ref, *, blk, nblk):
    i = pl.program_id(0)
    S = out_ref.shape[0]
    pos = jax.lax.broadcasted_iota(jnp.int32, (blk, S), 0) + i * blk
    st = starts_ref[...]
    en = ends_ref[...]
    on_mean = ((pos >= st) & (pos < en)).astype(jnp.float32)
    x = x_ref[...]
    dn = (((0,), (0,)), ((), ()))
    pm = jax.lax.dot_general(on_mean, x, dn, preferred_element_type=jnp.float32)

    @pl.when(i == 0)
    def _():
        out_ref[...] = pm

    @pl.when(i > 0)
    def _():
        out_ref[...] = out_ref[...] + pm


def _head_body(psum_ref, stc_ref, cls_ref, invc_ref, W_ref, b_ref, out_ref):
    sums = jnp.sum(psum_ref[...], axis=0) + stc_ref[...]
    mean = sums * invc_ref[...]
    pooled = jnp.concatenate([cls_ref[...], mean], axis=-1)
    out_ref[...] = (
        jnp.dot(pooled, W_ref[...], preferred_element_type=jnp.float32)
        + b_ref[...]
    )


def kernel(flat, cu_seqlens, W, b):
    T, D = flat.shape
    S = cu_seqlens.shape[0] - 1
    NL = W.shape[1]
    R = 64
    NW = NC * NS

    bounds = cu_seqlens[1:]
    starts = cu_seqlens[:-1]
    counts = (cu_seqlens[1:] - cu_seqlens[:-1]).astype(jnp.float32)
    invc = (1.0 / jnp.maximum(counts, 1.0)).reshape(S, 1)
    b2 = b.reshape(1, NL)
    starts2d = starts.reshape(1, S)
    ends2d = bounds.reshape(1, S)

    mesh = plsc.VectorSubcoreMesh(
        core_axis_name="c", subcore_axis_name="s",
        num_cores=NC, num_subcores=NS)
    sc = functools.partial(
        pl.kernel,
        out_type=[
            jax.ShapeDtypeStruct((NW, S, D), jnp.float32),
            jax.ShapeDtypeStruct((S, D), jnp.float32),
        ],
        mesh=mesh,
        compiler_params=pltpu.CompilerParams(needs_layout_passes=False),
        scratch_types=[
            pltpu.VMEM((S,), jnp.int32),
            pltpu.VMEM((S,), jnp.int32),
            pltpu.VMEM((R, D), jnp.float32),
            pltpu.VMEM((R, D), jnp.float32),
            pltpu.VMEM((S, D), jnp.float32),
            pltpu.SMEM((S + 1,), jnp.int32),
            pltpu.SemaphoreType.DMA,
            pltpu.SemaphoreType.DMA,
            pltpu.SemaphoreType.DMA,
        ],
    )(functools.partial(_sc_body, base0=TC_ROWS, T=T, D=D, S=S, R=R))
    psum, cls = sc(flat, bounds, starts)

    nblk = TC_ROWS // TC_BLK
    stc = pl.pallas_call(
        functools.partial(_tc_body, blk=TC_BLK, nblk=nblk),
        grid=(nblk,),
        in_specs=[
            pl.BlockSpec((1, S), lambda i: (0, 0)),
            pl.BlockSpec((1, S), lambda i: (0, 0)),
            pl.BlockSpec((TC_BLK, D), lambda i: (i, 0)),
        ],
        out_specs=pl.BlockSpec((S, D), lambda i: (0, 0)),
        out_shape=jax.ShapeDtypeStruct((S, D), jnp.float32),
        compiler_params=pltpu.CompilerParams(
            dimension_semantics=("arbitrary",),
        ),
    )(starts2d, ends2d, flat)

    out = pl.pallas_call(
        _head_body,
        in_specs=[
            pl.BlockSpec((NW, S, D), lambda: (0, 0, 0)),
            pl.BlockSpec((S, D), lambda: (0, 0)),
            pl.BlockSpec((S, D), lambda: (0, 0)),
            pl.BlockSpec((S, 1), lambda: (0, 0)),
            pl.BlockSpec((2 * D, NL), lambda: (0, 0)),
            pl.BlockSpec((1, NL), lambda: (0, 0)),
        ],
        out_specs=pl.BlockSpec((S, NL), lambda: (0, 0)),
        out_shape=jax.ShapeDtypeStruct((S, NL), jnp.float32),
    )(psum, stc, cls, invc, W, b2)
    return out


# speedup vs baseline: 1.8067x; 1.5016x over previous
import functools

import jax
import jax.numpy as jnp
from jax.experimental import pallas as pl
from jax.experimental.pallas import tpu as pltpu


def _body(starts_ref, ends_ref, invc_ref, xa_ref, xb_ref, W_ref, b_ref,
          out_ref, acc_mean, acc_cls, *, blk, nblk, half):
    i = pl.program_id(0)
    S = acc_mean.shape[0]
    st = starts_ref[...]
    en = ends_ref[...]
    dn = (((0,), (0,)), ((), ()))

    pos_a = jax.lax.broadcasted_iota(jnp.int32, (blk, S), 0) + i * blk
    pos_b = pos_a + half
    on_a = jnp.concatenate(
        [((pos_a >= st) & (pos_a < en)).astype(jnp.float32),
         (pos_a == st).astype(jnp.float32)], axis=-1)
    on_b = jnp.concatenate(
        [((pos_b >= st) & (pos_b < en)).astype(jnp.float32),
         (pos_b == st).astype(jnp.float32)], axis=-1)
    pa = jax.lax.dot_general(on_a, xa_ref[...], dn,
                             preferred_element_type=jnp.float32)
    pb = jax.lax.dot_general(on_b, xb_ref[...], dn,
                             preferred_element_type=jnp.float32)
    p = pa + pb

    @pl.when(i == 0)
    def _():
        acc_mean[...] = p[:S]
        acc_cls[...] = p[S:]

    @pl.when(i > 0)
    def _():
        acc_mean[...] = acc_mean[...] + p[:S]
        acc_cls[...] = acc_cls[...] + p[S:]

    @pl.when(i == nblk - 1)
    def _():
        mean = acc_mean[...] * invc_ref[...]
        pooled = jnp.concatenate([acc_cls[...], mean], axis=-1)
        out_ref[...] = (
            jnp.dot(pooled, W_ref[...], preferred_element_type=jnp.float32)
            + b_ref[...]
        )


def kernel(flat, cu_seqlens, W, b):
    T, D = flat.shape
    S = cu_seqlens.shape[0] - 1
    NL = W.shape[1]
    blk = 2048
    half = T // 2
    nblk = half // blk

    starts = cu_seqlens[:-1].reshape(1, S)
    ends = cu_seqlens[1:].reshape(1, S)
    counts = (cu_seqlens[1:] - cu_seqlens[:-1]).astype(jnp.float32)
    invc = (1.0 / jnp.maximum(counts, 1.0)).reshape(S, 1)
    b2 = b.reshape(1, NL)

    out = pl.pallas_call(
        functools.partial(_body, blk=blk, nblk=nblk, half=half),
        grid=(nblk,),
        in_specs=[
            pl.BlockSpec((1, S), lambda i: (0, 0)),
            pl.BlockSpec((1, S), lambda i: (0, 0)),
            pl.BlockSpec((S, 1), lambda i: (0, 0)),
            pl.BlockSpec((blk, D), lambda i: (i, 0)),
            pl.BlockSpec((blk, D), lambda i: (i + nblk, 0)),
            pl.BlockSpec((2 * D, NL), lambda i: (0, 0)),
            pl.BlockSpec((1, NL), lambda i: (0, 0)),
        ],
        out_specs=pl.BlockSpec((S, NL), lambda i: (0, 0)),
        out_shape=jax.ShapeDtypeStruct((S, NL), jnp.float32),
        scratch_shapes=[
            pltpu.VMEM((S, D), jnp.float32),
            pltpu.VMEM((S, D), jnp.float32),
        ],
        compiler_params=pltpu.CompilerParams(
            dimension_semantics=("arbitrary",),
        ),
    )(starts, ends, invc, flat, flat, W, b2)
    return out
